# Initial kernel scaffold; baseline (speedup 1.0000x reference)
import jax
import jax.numpy as jnp
from jax.experimental import pallas as pl


def kernel(X, edge_weight, bn_gamma, bn_beta, Wc, bc, W1, b1, W2, b2):
    raise NotImplementedError("write your pallas kernel here")


# speedup vs baseline: 2947.3966x; 2947.3966x over previous
import jax
import jax.numpy as jnp
from jax.experimental import pallas as pl

_F32 = jnp.float32
_HI = jax.lax.Precision.HIGHEST


def _prep_kernel(ew_ref, xt_ref, g_ref, beta_ref, w1r_ref, wc_ref,
                 t_ref, g_out_ref):
    ew = ew_ref[...]
    rows = [ew[:, i * (i + 1) // 2: i * (i + 1) // 2 + 64] for i in range(62)]
    rows.append(jnp.zeros((2, 64), _F32))
    tril = jnp.concatenate(rows, axis=0)
    ii = jax.lax.broadcasted_iota(jnp.int32, (64, 64), 0)
    jj = jax.lax.broadcasted_iota(jnp.int32, (64, 64), 1)
    tril = jnp.where(jj <= ii, tril, 0.0)
    A = jnp.where(ii >= jj, tril, tril.T)
    A = jnp.maximum(A, 0.0)
    d = jnp.sum(A, axis=1, keepdims=True)
    dinv = jax.lax.rsqrt(d + 1e-10)
    L = dinv * A * jnp.transpose(dinv)
    deg = jnp.sum(jnp.abs(L), axis=1, keepdims=True) + 1.0
    dis = jax.lax.rsqrt(deg)
    eye = jnp.where(ii == jj, 1.0, 0.0).astype(_F32)
    M = (dis * jnp.transpose(dis)) * (L + eye)
    M2 = jnp.dot(M, M, preferred_element_type=_F32, precision=_HI)
    xbn = xt_ref[...] * g_ref[...] + beta_ref[...]
    t_ref[...] = jnp.dot(M2, xbn, preferred_element_type=_F32, precision=_HI)
    g_out_ref[...] = jnp.dot(w1r_ref[...], wc_ref[...],
                             preferred_element_type=_F32, precision=_HI)


def _main_kernel(tb_ref, w1c_ref, w1_ref, bcrep_ref, b1_ref, w2t_ref, b2_ref,
                 o_ref):
    dn = (((1,), (1,)), ((), ()))
    out64 = jax.lax.dot_general(tb_ref[...], w1c_ref[...], dn,
                                preferred_element_type=_F32, precision=_HI)
    bct = jax.lax.dot_general(bcrep_ref[...], w1_ref[...], dn,
                              preferred_element_type=_F32, precision=_HI)
    h = jnp.maximum(out64 + bct + b1_ref[...], 0.0)
    o_ref[...] = jnp.dot(h, w2t_ref[...], preferred_element_type=_F32,
                         precision=_HI) + b2_ref[...]


def kernel(X, edge_weight, bn_gamma, bn_beta, Wc, bc, W1, b1, W2, b2):
    B, N, F = X.shape
    H = Wc.shape[0]
    O1 = W1.shape[0]
    C = W2.shape[0]

    ewp = jnp.pad(edge_weight, (0, 2048 - edge_weight.shape[0]))[None, :]
    g = (bn_gamma / jnp.sqrt(1.0 + 1e-5)).astype(_F32)
    gvec = jnp.tile(g, B)[None, :]
    bvec = jnp.tile(bn_beta.astype(_F32), B)[None, :]
    Xt = jnp.pad(X.transpose(1, 0, 2).reshape(N, B * F), ((0, 2), (0, 0)))
    W1r = W1.reshape(O1 * N, H)
    Wcp = jnp.pad(Wc, ((0, 0), (0, 128 - F)))

    T, G = pl.pallas_call(
        _prep_kernel,
        out_shape=[
            jax.ShapeDtypeStruct((64, B * F), _F32),
            jax.ShapeDtypeStruct((O1 * N, 128), _F32),
        ],
    )(ewp, Xt, gvec, bvec, W1r, Wcp)

    W1c = jnp.pad(G[:, :F].reshape(O1, N * F), ((0, 0), (0, 384 - N * F)))
    Tb2 = jnp.pad(T[:N].reshape(N, B, F).transpose(1, 0, 2).reshape(B, N * F),
                  ((0, 0), (0, 384 - N * F)))
    W1p = jnp.pad(W1, ((0, 0), (0, 2048 - N * H)))
    bcrep = jnp.pad(jnp.tile(bc, N), (0, 2048 - N * H))[None, :]
    b1p = b1[None, :]
    W2tp = jnp.pad(W2.T, ((0, 0), (0, 128 - C)))
    b2p = jnp.pad(b2, (0, 128 - C))[None, :]

    out = pl.pallas_call(
        _main_kernel,
        out_shape=jax.ShapeDtypeStruct((B, 128), _F32),
    )(Tb2, W1c, W1p, bcrep, b1p, W2tp, b2p)
    return out[:, :C]


# speedup vs baseline: 5806.9989x; 1.9702x over previous
import jax
import jax.numpy as jnp
from jax.experimental import pallas as pl

_F32 = jnp.float32
_HI = jax.lax.Precision.HIGHEST


def _iota2(shape, dim):
    return jax.lax.broadcasted_iota(jnp.int32, shape, dim)


def _dgcnn_kernel(x_ref, ew_ref, g_ref, beta_ref, wc_ref, bc_ref, w1_ref,
                  b1_ref, w2_ref, b2_ref, o_ref):
    NF = 310
    NH = 1984
    ew = ew_ref[...]
    rows = [ew[:, i * (i + 1) // 2: i * (i + 1) // 2 + 64] for i in range(61)]
    rows.append(jnp.concatenate([ew[:, 1891:1953], jnp.zeros((1, 2), _F32)],
                                axis=1))
    rows.append(jnp.zeros((2, 64), _F32))
    tril = jnp.concatenate(rows, axis=0)
    ii = _iota2((64, 64), 0)
    jj = _iota2((64, 64), 1)
    tril = jnp.where(jj <= ii, tril, 0.0)
    A = jnp.where(ii >= jj, tril, tril.T)
    A = jnp.maximum(A, 0.0)
    d = jnp.sum(A, axis=1, keepdims=True)
    dinv = jax.lax.rsqrt(d + 1e-10)
    L = dinv * A * jnp.transpose(dinv)
    deg = jnp.sum(jnp.abs(L), axis=1, keepdims=True) + 1.0
    dis = jax.lax.rsqrt(deg)
    eye = jnp.where(ii == jj, 1.0, 0.0).astype(_F32)
    M = (dis * jnp.transpose(dis)) * (L + eye)
    M2 = jnp.dot(M, M, preferred_element_type=_F32, precision=_HI)

    u5 = (_iota2((NF, 64), 0) // 5 == _iota2((NF, 64), 1)).astype(_F32)
    u5t = (_iota2((64, NF), 1) // 5 == _iota2((64, NF), 0)).astype(_F32)
    kq = jnp.dot(jnp.dot(u5, M2, preferred_element_type=_F32, precision=_HI),
                 u5t, preferred_element_type=_F32, precision=_HI)
    kq = kq * (_iota2((NF, NF), 0) % 5 == _iota2((NF, NF), 1) % 5).astype(_F32)

    v2 = (_iota2((5, NF), 1) % 5 == _iota2((5, NF), 0)).astype(_F32)
    gvec = jnp.dot(g_ref[...], v2, preferred_element_type=_F32, precision=_HI)
    bvec = jnp.dot(beta_ref[...], v2, preferred_element_type=_F32,
                   precision=_HI)
    xbn = x_ref[...] * gvec + bvec
    t2 = jnp.dot(xbn, kq, preferred_element_type=_F32, precision=_HI)

    v1 = (_iota2((NH, 32), 0) % 32 == _iota2((NH, 32), 1)).astype(_F32)
    wtile = jnp.dot(jnp.dot(v1, wc_ref[...], preferred_element_type=_F32,
                            precision=_HI), v2,
                    preferred_element_type=_F32, precision=_HI)
    bd = wtile * (_iota2((NH, NF), 0) // 32 == _iota2((NH, NF), 1) // 5
                  ).astype(_F32)
    w1c = jnp.dot(w1_ref[...], bd, preferred_element_type=_F32, precision=_HI)

    dn = (((1,), (1,)), ((), ()))
    out64 = jax.lax.dot_general(t2, w1c, dn,
                                preferred_element_type=_F32, precision=_HI)
    v1t = (_iota2((32, NH), 1) % 32 == _iota2((32, NH), 0)).astype(_F32)
    bcrep = jnp.dot(bc_ref[...], v1t, preferred_element_type=_F32,
                    precision=_HI)
    bct = jax.lax.dot_general(bcrep, w1_ref[...], dn,
                              preferred_element_type=_F32, precision=_HI)
    h = jnp.maximum(out64 + bct + b1_ref[...], 0.0)
    o_ref[...] = jax.lax.dot_general(h, w2_ref[...], dn,
                                     preferred_element_type=_F32,
                                     precision=_HI) + b2_ref[...]


def kernel(X, edge_weight, bn_gamma, bn_beta, Wc, bc, W1, b1, W2, b2):
    B, N, F = X.shape
    C = W2.shape[0]
    g = (bn_gamma / jnp.sqrt(1.0 + 1e-5)).astype(_F32)
    return pl.pallas_call(
        _dgcnn_kernel,
        out_shape=jax.ShapeDtypeStruct((B, C), _F32),
    )(X.reshape(B, N * F), edge_weight[None, :], g[None, :],
      bn_beta[None, :], Wc, bc[None, :], W1, b1[None, :], W2, b2[None, :])


# speedup vs baseline: 6092.6842x; 1.0492x over previous
import jax
import jax.numpy as jnp
from jax.experimental import pallas as pl

_F32 = jnp.float32
_HI = jax.lax.Precision.HIGHEST


def _iota2(shape, dim):
    return jax.lax.broadcasted_iota(jnp.int32, shape, dim)


def _dgcnn_kernel(x_ref, ew_ref, g_ref, beta_ref, wc_ref, bc_ref, w1_ref,
                  b1_ref, w2_ref, b2_ref, o_ref):
    NF = 310
    NH = 1984
    ew = ew_ref[...]
    rows = [ew[:, i * (i + 1) // 2: i * (i + 1) // 2 + 64] for i in range(61)]
    rows.append(jnp.concatenate([ew[:, 1891:1953], jnp.zeros((1, 2), _F32)],
                                axis=1))
    rows.append(jnp.zeros((2, 64), _F32))
    tril = jnp.concatenate(rows, axis=0)
    ii = _iota2((64, 64), 0)
    jj = _iota2((64, 64), 1)
    tril = jnp.where(jj <= ii, tril, 0.0)
    A = jnp.where(ii >= jj, tril, tril.T)
    A = jnp.maximum(A, 0.0)
    d = jnp.sum(A, axis=1, keepdims=True)
    dinv = jax.lax.rsqrt(d + 1e-10)
    L = dinv * A * jnp.transpose(dinv)
    deg = jnp.sum(jnp.abs(L), axis=1, keepdims=True) + 1.0
    dis = jax.lax.rsqrt(deg)
    eye = jnp.where(ii == jj, 1.0, 0.0).astype(_F32)
    M = (dis * jnp.transpose(dis)) * (L + eye)
    M2 = jnp.dot(M, M, preferred_element_type=_F32, precision=_HI)

    u5 = (_iota2((NF, 64), 0) // 5 == _iota2((NF, 64), 1)).astype(_F32)
    u5t = (_iota2((64, NF), 1) // 5 == _iota2((64, NF), 0)).astype(_F32)
    kq = jnp.dot(jnp.dot(u5, M2, preferred_element_type=_F32, precision=_HI),
                 u5t, preferred_element_type=_F32, precision=_HI)
    kq = kq * (_iota2((NF, NF), 0) % 5 == _iota2((NF, NF), 1) % 5).astype(_F32)

    v2 = (_iota2((5, NF), 1) % 5 == _iota2((5, NF), 0)).astype(_F32)
    v1 = (_iota2((NH, 32), 0) % 32 == _iota2((NH, 32), 1)).astype(_F32)
    wtile = jnp.dot(jnp.dot(v1, wc_ref[...], preferred_element_type=_F32,
                            precision=_HI), v2,
                    preferred_element_type=_F32, precision=_HI)
    bd = wtile * (_iota2((NH, NF), 0) // 32 == _iota2((NH, NF), 1) // 5
                  ).astype(_F32)
    w1c = jnp.dot(w1_ref[...], bd, preferred_element_type=_F32, precision=_HI)

    dn = (((1,), (1,)), ((), ()))
    w1ck = jax.lax.dot_general(kq, w1c, dn,
                               preferred_element_type=_F32, precision=_HI)
    gvec = jnp.dot(g_ref[...], v2, preferred_element_type=_F32, precision=_HI)
    bvec = jnp.dot(beta_ref[...], v2, preferred_element_type=_F32,
                   precision=_HI)
    w1ckg = jnp.transpose(gvec) * w1ck
    bbn = jnp.dot(bvec, w1ck, preferred_element_type=_F32, precision=_HI)

    out64 = jnp.dot(x_ref[...], w1ckg,
                    preferred_element_type=_F32, precision=_HI)
    v1t = (_iota2((32, NH), 1) % 32 == _iota2((32, NH), 0)).astype(_F32)
    bcrep = jnp.dot(bc_ref[...], v1t, preferred_element_type=_F32,
                    precision=_HI)
    bct = jax.lax.dot_general(bcrep, w1_ref[...], dn,
                              preferred_element_type=_F32, precision=_HI)
    h = jnp.maximum(out64 + (bbn + bct + b1_ref[...]), 0.0)
    o_ref[...] = jax.lax.dot_general(h, w2_ref[...], dn,
                                     preferred_element_type=_F32,
                                     precision=_HI) + b2_ref[...]


def kernel(X, edge_weight, bn_gamma, bn_beta, Wc, bc, W1, b1, W2, b2):
    B, N, F = X.shape
    C = W2.shape[0]
    g = (bn_gamma / jnp.sqrt(1.0 + 1e-5)).astype(_F32)
    return pl.pallas_call(
        _dgcnn_kernel,
        out_shape=jax.ShapeDtypeStruct((B, C), _F32),
    )(X.reshape(B, N * F), edge_weight[None, :], g[None, :],
      bn_beta[None, :], Wc, bc[None, :], W1, b1[None, :], W2, b2[None, :])


# speedup vs baseline: 10935.4248x; 1.7948x over previous
import jax
import jax.numpy as jnp
from jax.experimental import pallas as pl

_F32 = jnp.float32
_HI = jax.lax.Precision.DEFAULT


def _iota2(shape, dim):
    return jax.lax.broadcasted_iota(jnp.int32, shape, dim)


def _dot(a, b):
    return jax.lax.dot_general(a, b, (((1,), (0,)), ((), ())),
                               preferred_element_type=_F32, precision=_HI)


def _dott(a, b):
    return jax.lax.dot_general(a, b, (((1,), (1,)), ((), ())),
                               preferred_element_type=_F32, precision=_HI)


def _dgcnn_kernel(x_ref, ew_ref, g_ref, beta_ref, wc_ref, bc_ref, w1_ref,
                  b1_ref, w2_ref, b2_ref, o_ref):
    NF = 310
    NH = 1984
    ew = ew_ref[...]
    rows = [ew[:, i * (i + 1) // 2: i * (i + 1) // 2 + 64] for i in range(61)]
    rows.append(jnp.concatenate([ew[:, 1891:1953], jnp.zeros((1, 2), _F32)],
                                axis=1))
    rows.append(jnp.zeros((2, 64), _F32))
    tril = jnp.concatenate(rows, axis=0)
    ii = _iota2((64, 64), 0)
    jj = _iota2((64, 64), 1)
    tril = jnp.where(jj <= ii, tril, 0.0)
    A = jnp.where(ii >= jj, tril, tril.T)
    A = jnp.maximum(A, 0.0)
    d = jnp.sum(A, axis=1, keepdims=True)
    dinv = jax.lax.rsqrt(d + 1e-10)
    L = dinv * A * jnp.transpose(dinv)
    deg = jnp.sum(jnp.abs(L), axis=1, keepdims=True) + 1.0
    dis = jax.lax.rsqrt(deg)
    eye = jnp.where(ii == jj, 1.0, 0.0).astype(_F32)
    M = (dis * jnp.transpose(dis)) * (L + eye)
    M2 = _dot(M, M)

    u5 = (_iota2((NF, 64), 0) // 5 == _iota2((NF, 64), 1)).astype(_F32)
    u5t = (_iota2((64, NF), 1) // 5 == _iota2((64, NF), 0)).astype(_F32)
    kq = _dot(_dot(u5, M2), u5t)
    kq = kq * (_iota2((NF, NF), 0) % 5 == _iota2((NF, NF), 1) % 5).astype(_F32)

    v2 = (_iota2((5, NF), 1) % 5 == _iota2((5, NF), 0)).astype(_F32)
    v1 = (_iota2((NH, 32), 0) % 32 == _iota2((NH, 32), 1)).astype(_F32)
    wtile = _dot(_dot(v1, wc_ref[...]), v2)
    bd = wtile * (_iota2((NH, NF), 0) // 32 == _iota2((NH, NF), 1) // 5
                  ).astype(_F32)
    w1c = _dot(w1_ref[...], bd)

    dn = (((1,), (1,)), ((), ()))
    w1ck = _dott(kq, w1c)
    gvec = _dot(g_ref[...], v2)
    bvec = _dot(beta_ref[...], v2)
    w1ckg = jnp.transpose(gvec) * w1ck
    bbn = _dot(bvec, w1ck)

    out64 = _dot(x_ref[...], w1ckg)
    v1t = (_iota2((32, NH), 1) % 32 == _iota2((32, NH), 0)).astype(_F32)
    bcrep = _dot(bc_ref[...], v1t)
    bct = _dott(bcrep, w1_ref[...])
    h = jnp.maximum(out64 + (bbn + bct + b1_ref[...]), 0.0)
    o_ref[...] = _dott(h, w2_ref[...]) + b2_ref[...]


def kernel(X, edge_weight, bn_gamma, bn_beta, Wc, bc, W1, b1, W2, b2):
    B, N, F = X.shape
    C = W2.shape[0]
    g = (bn_gamma / jnp.sqrt(1.0 + 1e-5)).astype(_F32)
    return pl.pallas_call(
        _dgcnn_kernel,
        out_shape=jax.ShapeDtypeStruct((B, C), _F32),
    )(X.reshape(B, N * F), edge_weight[None, :], g[None, :],
      bn_beta[None, :], Wc, bc[None, :], W1, b1[None, :], W2, b2[None, :])
